# Initial kernel scaffold; baseline (speedup 1.0000x reference)
import jax
import jax.numpy as jnp
from jax.experimental import pallas as pl


def kernel(x, edge_index, W1, b1, g1, be1, Wc0, bc0, gc0, bec0, Wc1, bc1, gc1, bec1, Wp0, bp0, Wp1, bp1, Wp2, bp2):
    raise NotImplementedError("write your pallas kernel here")


# speedup vs baseline: 14.8802x; 14.8802x over previous
import functools

import jax
import jax.numpy as jnp
from jax import lax
from jax.experimental import pallas as pl
from jax.experimental.pallas import tpu as pltpu
from jax.experimental.pallas import tpu_sc as plsc

N = 10000
E = 320000
D = 128
H = 128
DH = 256
OUT = 40

NC = 2
NS = 16
NW = NC * NS
NPAD = 10240
NBLK = 80
NEB = 79
RPT = NPAD // NS
EPAD = NW * NEB * 128

_mesh = plsc.VectorSubcoreMesh(core_axis_name="c", subcore_axis_name="s")


@functools.partial(
    pl.kernel, mesh=_mesh,
    out_type=jax.ShapeDtypeStruct((NC * NPAD,), jnp.float32),
    scratch_types=[
        pltpu.VMEM((NEB, 128), jnp.int32),
        pltpu.VMEM((128,), jnp.float32),
        pltpu.VMEM_SHARED((NPAD,), jnp.float32),
    ],
)
def _deg_kernel(dst_hbm, ones_hbm, zeros1_hbm, out_hbm, idx_v, ones_v, hist_sh):
    c = lax.axis_index("c")
    s = lax.axis_index("s")
    w = c * NS + s
    pltpu.sync_copy(zeros1_hbm.at[pl.ds(s * RPT, RPT)], hist_sh.at[pl.ds(s * RPT, RPT)])
    pltpu.sync_copy(ones_hbm, ones_v)
    pltpu.sync_copy(dst_hbm.at[w], idx_v)
    plsc.subcore_barrier()

    def body(j, carry):
        pltpu.sync_copy(ones_v, hist_sh.at[idx_v.at[j]], add=True)
        return carry

    lax.fori_loop(0, NEB, body, 0)
    plsc.subcore_barrier()
    pltpu.sync_copy(hist_sh.at[pl.ds(s * RPT, RPT)],
                    out_hbm.at[pl.ds(c * NPAD + s * RPT, RPT)])


@functools.partial(
    pl.kernel, mesh=_mesh,
    out_type=jax.ShapeDtypeStruct((NC, NPAD, H), jnp.float32),
    scratch_types=[
        pltpu.VMEM((NEB, 128), jnp.int32),
        pltpu.VMEM((NEB, 128), jnp.int32),
        pltpu.VMEM((128, H), jnp.float32),
        pltpu.VMEM_SHARED((NPAD, H), jnp.float32),
        pltpu.SemaphoreType.DMA,
    ],
)
def _scat_kernel(y_hbm, srcw_hbm, dstw_hbm, zeros2_hbm, out_hbm,
                 src_v, dst_v, rows_v, acc_sh, gsem):
    c = lax.axis_index("c")
    s = lax.axis_index("s")
    w = c * NS + s
    pltpu.sync_copy(zeros2_hbm.at[pl.ds(s * RPT, RPT)], acc_sh.at[pl.ds(s * RPT, RPT)])
    pltpu.sync_copy(srcw_hbm.at[w], src_v)
    pltpu.sync_copy(dstw_hbm.at[w], dst_v)
    plsc.subcore_barrier()

    def body(j, carry):
        pltpu.async_copy(y_hbm.at[src_v.at[j]], rows_v, gsem).wait()
        pltpu.sync_copy(rows_v, acc_sh.at[dst_v.at[j]], add=True)
        return carry

    lax.fori_loop(0, NEB, body, 0)
    plsc.subcore_barrier()
    pltpu.sync_copy(acc_sh.at[pl.ds(s * RPT, RPT)], out_hbm.at[c, pl.ds(s * RPT, RPT)])


def _stage0_body(deg_ref, x_ref, w_ref, y_ref, dinv_ref):
    deg = deg_ref[0, 0, 0, :] + deg_ref[1, 0, 0, :] + 1.0
    dinv = lax.rsqrt(deg)
    xw = jnp.dot(x_ref[...], w_ref[...], preferred_element_type=jnp.float32)
    y_ref[...] = xw * dinv[:, None]
    dinv_ref[0, 0, :] = dinv


_stage0 = pl.pallas_call(
    _stage0_body,
    grid=(NBLK,),
    in_specs=[
        pl.BlockSpec((2, 1, 1, 128), lambda i: (0, i, 0, 0)),
        pl.BlockSpec((128, 128), lambda i: (i, 0)),
        pl.BlockSpec((128, 128), lambda i: (0, 0)),
    ],
    out_specs=[
        pl.BlockSpec((128, 128), lambda i: (i, 0)),
        pl.BlockSpec((1, 1, 128), lambda i: (i, 0, 0)),
    ],
    out_shape=[
        jax.ShapeDtypeStruct((NPAD, H), jnp.float32),
        jax.ShapeDtypeStruct((NBLK, 1, 128), jnp.float32),
    ],
)


def _mid_body(p_ref, y_ref, dinv_ref, b_ref, g_ref, be_ref, w_ref, o_ref):
    z = p_ref[0] + p_ref[1] + y_ref[...]
    d = dinv_ref[0, 0, :]
    gcn = z * d[:, None] + b_ref[...][None, :]
    a = jnp.maximum(gcn, 0.0)
    m = jnp.mean(a, axis=-1, keepdims=True)
    v = jnp.mean((a - m) ** 2, axis=-1, keepdims=True)
    h = (a - m) / jnp.sqrt(v + 1e-5) * g_ref[...][None, :] + be_ref[...][None, :]
    o_ref[...] = jnp.dot(h, w_ref[...], preferred_element_type=jnp.float32) * d[:, None]


_stage_mid = pl.pallas_call(
    _mid_body,
    grid=(NBLK,),
    in_specs=[
        pl.BlockSpec((2, 128, 128), lambda i: (0, i, 0)),
        pl.BlockSpec((128, 128), lambda i: (i, 0)),
        pl.BlockSpec((1, 1, 128), lambda i: (i, 0, 0)),
        pl.BlockSpec((128,), lambda i: (0,)),
        pl.BlockSpec((128,), lambda i: (0,)),
        pl.BlockSpec((128,), lambda i: (0,)),
        pl.BlockSpec((128, 128), lambda i: (0, 0)),
    ],
    out_specs=pl.BlockSpec((128, 128), lambda i: (i, 0)),
    out_shape=jax.ShapeDtypeStruct((NPAD, H), jnp.float32),
)


def _s3_body(p_ref, y_ref, dinv_ref, b_ref, g_ref, be_ref,
             wp0_ref, bp0_ref, wp1_ref, bp1_ref, wp2_ref, bp2_ref,
             emb_ref, lp_ref):
    z = p_ref[0] + p_ref[1] + y_ref[...]
    d = dinv_ref[0, 0, :]
    emb = z * d[:, None] + b_ref[...][None, :]
    emb_ref[...] = emb
    a = jnp.maximum(emb, 0.0)
    m = jnp.mean(a, axis=-1, keepdims=True)
    v = jnp.mean((a - m) ** 2, axis=-1, keepdims=True)
    h = (a - m) / jnp.sqrt(v + 1e-5) * g_ref[...][None, :] + be_ref[...][None, :]
    t = jnp.dot(h, wp0_ref[...], preferred_element_type=jnp.float32) + bp0_ref[...][None, :]
    t = jnp.dot(t, wp1_ref[...], preferred_element_type=jnp.float32) + bp1_ref[...][None, :]
    t = jnp.dot(t, wp2_ref[...], preferred_element_type=jnp.float32) + bp2_ref[...][None, :]
    mx = jnp.max(t, axis=-1, keepdims=True)
    lse = mx + jnp.log(jnp.sum(jnp.exp(t - mx), axis=-1, keepdims=True))
    lp_ref[...] = t - lse


_stage3 = pl.pallas_call(
    _s3_body,
    grid=(NBLK,),
    in_specs=[
        pl.BlockSpec((2, 128, 128), lambda i: (0, i, 0)),
        pl.BlockSpec((128, 128), lambda i: (i, 0)),
        pl.BlockSpec((1, 1, 128), lambda i: (i, 0, 0)),
        pl.BlockSpec((128,), lambda i: (0,)),
        pl.BlockSpec((128,), lambda i: (0,)),
        pl.BlockSpec((128,), lambda i: (0,)),
        pl.BlockSpec((128, DH), lambda i: (0, 0)),
        pl.BlockSpec((DH,), lambda i: (0,)),
        pl.BlockSpec((DH, DH // 2), lambda i: (0, 0)),
        pl.BlockSpec((DH // 2,), lambda i: (0,)),
        pl.BlockSpec((DH // 2, OUT), lambda i: (0, 0)),
        pl.BlockSpec((OUT,), lambda i: (0,)),
    ],
    out_specs=[
        pl.BlockSpec((128, 128), lambda i: (i, 0)),
        pl.BlockSpec((128, OUT), lambda i: (i, 0)),
    ],
    out_shape=[
        jax.ShapeDtypeStruct((NPAD, H), jnp.float32),
        jax.ShapeDtypeStruct((NPAD, OUT), jnp.float32),
    ],
)


def kernel(x, edge_index, W1, b1, g1, be1, Wc0, bc0, gc0, bec0,
           Wc1, bc1, gc1, bec1, Wp0, bp0, Wp1, bp1, Wp2, bp2):
    src = edge_index[0]
    dst = edge_index[1]
    pad = (N + (jnp.arange(EPAD - E, dtype=jnp.int32) % (NPAD - N))).astype(jnp.int32)
    srcp = jnp.concatenate([src, pad]).reshape(NW, NEB, 128)
    dstp = jnp.concatenate([dst, pad]).reshape(NW, NEB, 128)
    ones128 = jnp.ones((128,), jnp.float32)
    zeros1 = jnp.zeros((NPAD,), jnp.float32)
    zeros2 = jnp.zeros((NPAD, H), jnp.float32)

    degp = _deg_kernel(dstp, ones128, zeros1)
    xpad = jnp.pad(x, ((0, NPAD - N), (0, 0)))
    y0, dinv2d = _stage0(degp.reshape(NC, NBLK, 1, 128), xpad, W1)
    p0 = _scat_kernel(y0, srcp, dstp, zeros2)
    y1 = _stage_mid(p0, y0, dinv2d, b1, g1, be1, Wc0)
    p1 = _scat_kernel(y1, srcp, dstp, zeros2)
    y2 = _stage_mid(p1, y1, dinv2d, bc0, gc0, bec0, Wc1)
    p2 = _scat_kernel(y2, srcp, dstp, zeros2)
    emb, logp = _stage3(p2, y2, dinv2d, bc1, gc1, bec1,
                        Wp0, bp0, Wp1, bp1, Wp2, bp2)
    return emb[:N], logp[:N]


# speedup vs baseline: 19.9671x; 1.3419x over previous
import functools

import jax
import jax.numpy as jnp
from jax import lax
from jax.experimental import pallas as pl
from jax.experimental.pallas import tpu as pltpu
from jax.experimental.pallas import tpu_sc as plsc

N = 10000
E = 320000
D = 128
H = 128
DH = 256
OUT = 40

NC = 2
NS = 16
NW = NC * NS
NPAD = 10112
NBLK = 79
RPT = NPAD // NS
NPD = 10240
RPTD = NPD // NS
EPAD = NW * NBLK * 128
NEB = NBLK

_mesh = plsc.VectorSubcoreMesh(core_axis_name="c", subcore_axis_name="s")


@functools.partial(
    pl.kernel, mesh=_mesh,
    out_type=jax.ShapeDtypeStruct((NC * NPD,), jnp.float32),
    scratch_types=[
        pltpu.VMEM((NEB, 128), jnp.int32),
        pltpu.VMEM((128,), jnp.float32),
        pltpu.VMEM_SHARED((NPD,), jnp.float32),
    ],
)
def _deg_kernel(dst_hbm, ones_hbm, zeros1_hbm, out_hbm, idx_v, ones_v, hist_sh):
    c = lax.axis_index("c")
    s = lax.axis_index("s")
    w = c * NS + s
    pltpu.sync_copy(zeros1_hbm.at[pl.ds(s * RPTD, RPTD)], hist_sh.at[pl.ds(s * RPTD, RPTD)])
    pltpu.sync_copy(ones_hbm, ones_v)
    pltpu.sync_copy(dst_hbm.at[w], idx_v)
    plsc.subcore_barrier()

    def body(j, carry):
        pltpu.sync_copy(ones_v, hist_sh.at[idx_v.at[j]], add=True)
        return carry

    lax.fori_loop(0, NEB, body, 0)
    plsc.subcore_barrier()
    pltpu.sync_copy(hist_sh.at[pl.ds(s * RPTD, RPTD)],
                    out_hbm.at[pl.ds(c * NPD + s * RPTD, RPTD)])


@functools.partial(
    pl.kernel, mesh=_mesh,
    out_type=jax.ShapeDtypeStruct((NC, NPAD, H), jnp.float32),
    scratch_types=[
        pltpu.VMEM((NEB, 128), jnp.int32),
        pltpu.VMEM((2, 128), jnp.int32),
        pltpu.VMEM((2, 128, H), jnp.float32),
        pltpu.VMEM_SHARED((NPAD, H), jnp.float32),
        pltpu.SemaphoreType.DMA,
        pltpu.SemaphoreType.DMA,
    ],
)
def _scat_kernel(y_hbm, srcw_hbm, dstw_hbm, zeros2_hbm, out_hbm,
                 src_v, dstc_v, rows_v, acc_sh, gsem0, gsem1):
    c = lax.axis_index("c")
    s = lax.axis_index("s")
    w = c * NS + s
    pltpu.sync_copy(zeros2_hbm.at[pl.ds(s * RPT, RPT)], acc_sh.at[pl.ds(s * RPT, RPT)])
    pltpu.sync_copy(srcw_hbm.at[w], src_v)
    plsc.subcore_barrier()

    def _start(j, b, sem):
        pltpu.async_copy(y_hbm.at[src_v.at[j]], rows_v.at[b], sem)
        pltpu.async_copy(dstw_hbm.at[w, j], dstc_v.at[b], sem)

    def _wait(b, sem):
        pltpu.make_async_copy(y_hbm.at[src_v.at[0]], rows_v.at[b], sem).wait()
        pltpu.make_async_copy(dstw_hbm.at[0, 0], dstc_v.at[b], sem).wait()

    def _scat(b):
        pltpu.sync_copy(rows_v.at[b], acc_sh.at[dstc_v.at[b]], add=True)

    _start(0, 0, gsem0)

    def body(i, carry):
        j = 2 * i
        _start(j + 1, 1, gsem1)
        _wait(0, gsem0)
        _scat(0)
        _start(j + 2, 0, gsem0)
        _wait(1, gsem1)
        _scat(1)
        return carry

    lax.fori_loop(0, (NEB - 1) // 2, body, 0)
    _wait(0, gsem0)
    _scat(0)
    plsc.subcore_barrier()
    pltpu.sync_copy(acc_sh.at[pl.ds(s * RPT, RPT)], out_hbm.at[c, pl.ds(s * RPT, RPT)])


def _stage0_body(deg_ref, x_ref, w_ref, y_ref, dinv_ref):
    deg = deg_ref[0, 0, 0, :] + deg_ref[1, 0, 0, :] + 1.0
    dinv = lax.rsqrt(deg)
    xw = jnp.dot(x_ref[...], w_ref[...], preferred_element_type=jnp.float32)
    y_ref[...] = xw * dinv[:, None]
    dinv_ref[0, 0, :] = dinv


_stage0 = pl.pallas_call(
    _stage0_body,
    grid=(NBLK,),
    in_specs=[
        pl.BlockSpec((2, 1, 1, 128), lambda i: (0, i, 0, 0)),
        pl.BlockSpec((128, 128), lambda i: (i, 0)),
        pl.BlockSpec((128, 128), lambda i: (0, 0)),
    ],
    out_specs=[
        pl.BlockSpec((128, 128), lambda i: (i, 0)),
        pl.BlockSpec((1, 1, 128), lambda i: (i, 0, 0)),
    ],
    out_shape=[
        jax.ShapeDtypeStruct((NPAD, H), jnp.float32),
        jax.ShapeDtypeStruct((NBLK, 1, 128), jnp.float32),
    ],
)


def _mid_body(p_ref, y_ref, dinv_ref, b_ref, g_ref, be_ref, w_ref, o_ref):
    z = p_ref[0] + p_ref[1] + y_ref[...]
    d = dinv_ref[0, 0, :]
    gcn = z * d[:, None] + b_ref[...][None, :]
    a = jnp.maximum(gcn, 0.0)
    m = jnp.mean(a, axis=-1, keepdims=True)
    v = jnp.mean((a - m) ** 2, axis=-1, keepdims=True)
    h = (a - m) / jnp.sqrt(v + 1e-5) * g_ref[...][None, :] + be_ref[...][None, :]
    o_ref[...] = jnp.dot(h, w_ref[...], preferred_element_type=jnp.float32) * d[:, None]


_stage_mid = pl.pallas_call(
    _mid_body,
    grid=(NBLK,),
    in_specs=[
        pl.BlockSpec((2, 128, 128), lambda i: (0, i, 0)),
        pl.BlockSpec((128, 128), lambda i: (i, 0)),
        pl.BlockSpec((1, 1, 128), lambda i: (i, 0, 0)),
        pl.BlockSpec((128,), lambda i: (0,)),
        pl.BlockSpec((128,), lambda i: (0,)),
        pl.BlockSpec((128,), lambda i: (0,)),
        pl.BlockSpec((128, 128), lambda i: (0, 0)),
    ],
    out_specs=pl.BlockSpec((128, 128), lambda i: (i, 0)),
    out_shape=jax.ShapeDtypeStruct((NPAD, H), jnp.float32),
)


def _s3_body(p_ref, y_ref, dinv_ref, b_ref, g_ref, be_ref,
             wp0_ref, bp0_ref, wp1_ref, bp1_ref, wp2_ref, bp2_ref,
             emb_ref, lp_ref):
    z = p_ref[0] + p_ref[1] + y_ref[...]
    d = dinv_ref[0, 0, :]
    emb = z * d[:, None] + b_ref[...][None, :]
    emb_ref[...] = emb
    a = jnp.maximum(emb, 0.0)
    m = jnp.mean(a, axis=-1, keepdims=True)
    v = jnp.mean((a - m) ** 2, axis=-1, keepdims=True)
    h = (a - m) / jnp.sqrt(v + 1e-5) * g_ref[...][None, :] + be_ref[...][None, :]
    t = jnp.dot(h, wp0_ref[...], preferred_element_type=jnp.float32) + bp0_ref[...][None, :]
    t = jnp.dot(t, wp1_ref[...], preferred_element_type=jnp.float32) + bp1_ref[...][None, :]
    t = jnp.dot(t, wp2_ref[...], preferred_element_type=jnp.float32) + bp2_ref[...][None, :]
    mx = jnp.max(t, axis=-1, keepdims=True)
    lse = mx + jnp.log(jnp.sum(jnp.exp(t - mx), axis=-1, keepdims=True))
    lp_ref[...] = t - lse


_stage3 = pl.pallas_call(
    _s3_body,
    grid=(NBLK,),
    in_specs=[
        pl.BlockSpec((2, 128, 128), lambda i: (0, i, 0)),
        pl.BlockSpec((128, 128), lambda i: (i, 0)),
        pl.BlockSpec((1, 1, 128), lambda i: (i, 0, 0)),
        pl.BlockSpec((128,), lambda i: (0,)),
        pl.BlockSpec((128,), lambda i: (0,)),
        pl.BlockSpec((128,), lambda i: (0,)),
        pl.BlockSpec((128, DH), lambda i: (0, 0)),
        pl.BlockSpec((DH,), lambda i: (0,)),
        pl.BlockSpec((DH, DH // 2), lambda i: (0, 0)),
        pl.BlockSpec((DH // 2,), lambda i: (0,)),
        pl.BlockSpec((DH // 2, OUT), lambda i: (0, 0)),
        pl.BlockSpec((OUT,), lambda i: (0,)),
    ],
    out_specs=[
        pl.BlockSpec((128, 128), lambda i: (i, 0)),
        pl.BlockSpec((128, OUT), lambda i: (i, 0)),
    ],
    out_shape=[
        jax.ShapeDtypeStruct((NPAD, H), jnp.float32),
        jax.ShapeDtypeStruct((NPAD, OUT), jnp.float32),
    ],
)


def kernel(x, edge_index, W1, b1, g1, be1, Wc0, bc0, gc0, bec0,
           Wc1, bc1, gc1, bec1, Wp0, bp0, Wp1, bp1, Wp2, bp2):
    src = edge_index[0]
    dst = edge_index[1]
    pad = (N + (jnp.arange(EPAD - E, dtype=jnp.int32) % (NPAD - N))).astype(jnp.int32)
    srcp = jnp.concatenate([src, pad]).reshape(NW, NEB, 128)
    dstp128 = jnp.concatenate([dst, pad]).reshape(NW, NEB, 128)
    ones128 = jnp.ones((128,), jnp.float32)
    zeros1 = jnp.zeros((NPD,), jnp.float32)
    zeros2 = jnp.zeros((NPAD, H), jnp.float32)

    degp = _deg_kernel(dstp128, ones128, zeros1)
    deg4 = degp.reshape(NC, NPD // 128, 1, 128)[:, :NBLK]
    xpad = jnp.pad(x, ((0, NPAD - N), (0, 0)))
    y0, dinv2d = _stage0(deg4, xpad, W1)
    p0 = _scat_kernel(y0, srcp, dstp128, zeros2)
    y1 = _stage_mid(p0, y0, dinv2d, b1, g1, be1, Wc0)
    p1 = _scat_kernel(y1, srcp, dstp128, zeros2)
    y2 = _stage_mid(p1, y1, dinv2d, bc0, gc0, bec0, Wc1)
    p2 = _scat_kernel(y2, srcp, dstp128, zeros2)
    emb, logp = _stage3(p2, y2, dinv2d, bc1, gc1, bec1,
                        Wp0, bp0, Wp1, bp1, Wp2, bp2)
    return emb[:N], logp[:N]


# speedup vs baseline: 27.5306x; 1.3788x over previous
import functools

import jax
import jax.numpy as jnp
from jax import lax
from jax.experimental import pallas as pl
from jax.experimental.pallas import tpu as pltpu
from jax.experimental.pallas import tpu_sc as plsc

N = 10000
E = 320000
D = 128
H = 128
DH = 256
OUT = 40

NC = 2
NS = 16
NW = NC * NS
NACC = 10112
RPT = NACC // NS
NTC = 10240
RPTD = NTC // NS
RBLK = 1280
GBLK = NTC // RBLK
EPAD = NW * 79 * 128
NEB = 79

_mesh = plsc.VectorSubcoreMesh(core_axis_name="c", subcore_axis_name="s")


@functools.partial(
    pl.kernel, mesh=_mesh,
    out_type=jax.ShapeDtypeStruct((NC * NTC,), jnp.float32),
    scratch_types=[
        pltpu.VMEM((NEB, 128), jnp.int32),
        pltpu.VMEM((128,), jnp.float32),
        pltpu.VMEM_SHARED((NTC,), jnp.float32),
    ],
)
def _deg_kernel(dst_hbm, ones_hbm, zeros1_hbm, out_hbm, idx_v, ones_v, hist_sh):
    c = lax.axis_index("c")
    s = lax.axis_index("s")
    w = c * NS + s
    pltpu.sync_copy(zeros1_hbm.at[pl.ds(s * RPTD, RPTD)], hist_sh.at[pl.ds(s * RPTD, RPTD)])
    pltpu.sync_copy(ones_hbm, ones_v)
    pltpu.sync_copy(dst_hbm.at[w], idx_v)
    plsc.subcore_barrier()

    def body(j, carry):
        pltpu.sync_copy(ones_v, hist_sh.at[idx_v.at[j]], add=True)
        return carry

    lax.fori_loop(0, NEB, body, 0)
    plsc.subcore_barrier()
    pltpu.sync_copy(hist_sh.at[pl.ds(s * RPTD, RPTD)],
                    out_hbm.at[pl.ds(c * NTC + s * RPTD, RPTD)])


@functools.partial(
    pl.kernel, mesh=_mesh,
    out_type=jax.ShapeDtypeStruct((NC, NTC, H), jnp.float32),
    scratch_types=[
        pltpu.VMEM((NEB, 128), jnp.int32),
        pltpu.VMEM((2, 128), jnp.int32),
        pltpu.VMEM((2, 128, H), jnp.float32),
        pltpu.VMEM_SHARED((NACC, H), jnp.float32),
        pltpu.SemaphoreType.DMA,
        pltpu.SemaphoreType.DMA,
    ],
)
def _scat_kernel(y_hbm, srcw_hbm, dstw_hbm, zeros2_hbm, out_hbm,
                 src_v, dstc_v, rows_v, acc_sh, gsem0, gsem1):
    c = lax.axis_index("c")
    s = lax.axis_index("s")
    w = c * NS + s
    pltpu.sync_copy(zeros2_hbm.at[pl.ds(s * RPT, RPT)], acc_sh.at[pl.ds(s * RPT, RPT)])
    pltpu.sync_copy(srcw_hbm.at[w], src_v)
    plsc.subcore_barrier()

    def _start(j, b, sem):
        pltpu.async_copy(y_hbm.at[src_v.at[j]], rows_v.at[b], sem)
        pltpu.async_copy(dstw_hbm.at[w, j], dstc_v.at[b], sem)

    def _wait(b, sem):
        pltpu.make_async_copy(y_hbm.at[src_v.at[0]], rows_v.at[b], sem).wait()
        pltpu.make_async_copy(dstw_hbm.at[0, 0], dstc_v.at[b], sem).wait()

    def _scat(b):
        pltpu.sync_copy(rows_v.at[b], acc_sh.at[dstc_v.at[b]], add=True)

    _start(0, 0, gsem0)

    def body(i, carry):
        j = 2 * i
        _start(j + 1, 1, gsem1)
        _wait(0, gsem0)
        _scat(0)
        _start(j + 2, 0, gsem0)
        _wait(1, gsem1)
        _scat(1)
        return carry

    lax.fori_loop(0, (NEB - 1) // 2, body, 0)
    _wait(0, gsem0)
    _scat(0)
    plsc.subcore_barrier()
    pltpu.sync_copy(acc_sh.at[pl.ds(s * RPT, RPT)], out_hbm.at[c, pl.ds(s * RPT, RPT)])


def _dinv_body(deg_ref, dinv_ref):
    deg = deg_ref[0, :] + deg_ref[1, :] + 1.0
    dinv_ref[...] = lax.rsqrt(deg)[:, None]


_dinv_kernel = pl.pallas_call(
    _dinv_body,
    grid=(GBLK,),
    in_specs=[pl.BlockSpec((2, RBLK), lambda i: (0, i))],
    out_specs=pl.BlockSpec((RBLK, 1), lambda i: (i, 0)),
    out_shape=jax.ShapeDtypeStruct((NTC, 1), jnp.float32),
)


def _stage0_body(dinv_ref, x_ref, w_ref, y_ref):
    xw = jnp.dot(x_ref[...], w_ref[...], preferred_element_type=jnp.float32)
    y_ref[...] = xw * dinv_ref[...]


_stage0 = pl.pallas_call(
    _stage0_body,
    grid=(GBLK,),
    in_specs=[
        pl.BlockSpec((RBLK, 1), lambda i: (i, 0)),
        pl.BlockSpec((RBLK, 128), lambda i: (i, 0)),
        pl.BlockSpec((128, 128), lambda i: (0, 0)),
    ],
    out_specs=pl.BlockSpec((RBLK, 128), lambda i: (i, 0)),
    out_shape=jax.ShapeDtypeStruct((NTC, H), jnp.float32),
)


def _ln_relu(gcn, g, be):
    a = jnp.maximum(gcn, 0.0)
    m = jnp.mean(a, axis=-1, keepdims=True)
    v = jnp.mean((a - m) ** 2, axis=-1, keepdims=True)
    return (a - m) / jnp.sqrt(v + 1e-5) * g[None, :] + be[None, :]


def _mid_body(p_ref, y_ref, dinv_ref, b_ref, g_ref, be_ref, w_ref, o_ref):
    d = dinv_ref[...]
    z = p_ref[0] + p_ref[1] + y_ref[...]
    gcn = z * d + b_ref[...][None, :]
    h = _ln_relu(gcn, g_ref[...], be_ref[...])
    o_ref[...] = jnp.dot(h, w_ref[...], preferred_element_type=jnp.float32) * d


_stage_mid = pl.pallas_call(
    _mid_body,
    grid=(GBLK,),
    in_specs=[
        pl.BlockSpec((2, RBLK, 128), lambda i: (0, i, 0)),
        pl.BlockSpec((RBLK, 128), lambda i: (i, 0)),
        pl.BlockSpec((RBLK, 1), lambda i: (i, 0)),
        pl.BlockSpec((128,), lambda i: (0,)),
        pl.BlockSpec((128,), lambda i: (0,)),
        pl.BlockSpec((128,), lambda i: (0,)),
        pl.BlockSpec((128, 128), lambda i: (0, 0)),
    ],
    out_specs=pl.BlockSpec((RBLK, 128), lambda i: (i, 0)),
    out_shape=jax.ShapeDtypeStruct((NTC, H), jnp.float32),
)


def _s3_body(p_ref, y_ref, dinv_ref, b_ref, g_ref, be_ref,
             wp0_ref, bp0_ref, wp1_ref, bp1_ref, wp2_ref, bp2_ref,
             emb_ref, lp_ref):
    d = dinv_ref[...]
    z = p_ref[0] + p_ref[1] + y_ref[...]
    emb = z * d + b_ref[...][None, :]
    emb_ref[...] = emb
    h = _ln_relu(emb, g_ref[...], be_ref[...])
    t = jnp.dot(h, wp0_ref[...], preferred_element_type=jnp.float32) + bp0_ref[...][None, :]
    t = jnp.dot(t, wp1_ref[...], preferred_element_type=jnp.float32) + bp1_ref[...][None, :]
    t = jnp.dot(t, wp2_ref[...], preferred_element_type=jnp.float32) + bp2_ref[...][None, :]
    mx = jnp.max(t, axis=-1, keepdims=True)
    lse = mx + jnp.log(jnp.sum(jnp.exp(t - mx), axis=-1, keepdims=True))
    lp_ref[...] = t - lse


_stage3 = pl.pallas_call(
    _s3_body,
    grid=(GBLK,),
    in_specs=[
        pl.BlockSpec((2, RBLK, 128), lambda i: (0, i, 0)),
        pl.BlockSpec((RBLK, 128), lambda i: (i, 0)),
        pl.BlockSpec((RBLK, 1), lambda i: (i, 0)),
        pl.BlockSpec((128,), lambda i: (0,)),
        pl.BlockSpec((128,), lambda i: (0,)),
        pl.BlockSpec((128,), lambda i: (0,)),
        pl.BlockSpec((128, DH), lambda i: (0, 0)),
        pl.BlockSpec((DH,), lambda i: (0,)),
        pl.BlockSpec((DH, DH // 2), lambda i: (0, 0)),
        pl.BlockSpec((DH // 2,), lambda i: (0,)),
        pl.BlockSpec((DH // 2, OUT), lambda i: (0, 0)),
        pl.BlockSpec((OUT,), lambda i: (0,)),
    ],
    out_specs=[
        pl.BlockSpec((RBLK, 128), lambda i: (i, 0)),
        pl.BlockSpec((RBLK, OUT), lambda i: (i, 0)),
    ],
    out_shape=[
        jax.ShapeDtypeStruct((NTC, H), jnp.float32),
        jax.ShapeDtypeStruct((NTC, OUT), jnp.float32),
    ],
)


def kernel(x, edge_index, W1, b1, g1, be1, Wc0, bc0, gc0, bec0,
           Wc1, bc1, gc1, bec1, Wp0, bp0, Wp1, bp1, Wp2, bp2):
    src = edge_index[0]
    dst = edge_index[1]
    pad = (N + (jnp.arange(EPAD - E, dtype=jnp.int32) % (NACC - N))).astype(jnp.int32)
    srcp = jnp.concatenate([src, pad]).reshape(NW, NEB, 128)
    dstp128 = jnp.concatenate([dst, pad]).reshape(NW, NEB, 128)
    ones128 = jnp.ones((128,), jnp.float32)
    zeros1 = jnp.zeros((NTC,), jnp.float32)
    zeros2 = jnp.zeros((NACC, H), jnp.float32)

    degp = _deg_kernel(dstp128, ones128, zeros1)
    dinv = _dinv_kernel(degp.reshape(NC, NTC))
    xpad = jnp.pad(x, ((0, NTC - N), (0, 0)))
    y0 = _stage0(dinv, xpad, W1)
    p0 = _scat_kernel(y0, srcp, dstp128, zeros2)
    y1 = _stage_mid(p0, y0, dinv, b1, g1, be1, Wc0)
    p1 = _scat_kernel(y1, srcp, dstp128, zeros2)
    y2 = _stage_mid(p1, y1, dinv, bc0, gc0, bec0, Wc1)
    p2 = _scat_kernel(y2, srcp, dstp128, zeros2)
    emb, logp = _stage3(p2, y2, dinv, bc1, gc1, bec1,
                        Wp0, bp0, Wp1, bp1, Wp2, bp2)
    return emb[:N], logp[:N]


# speedup vs baseline: 28.4084x; 1.0319x over previous
import functools

import jax
import jax.numpy as jnp
from jax import lax
from jax.experimental import pallas as pl
from jax.experimental.pallas import tpu as pltpu
from jax.experimental.pallas import tpu_sc as plsc

N = 10000
E = 320000
D = 128
H = 128
DH = 256
OUT = 40

NC = 2
NS = 16
NW = NC * NS
NACC = 10112
RPT = NACC // NS
NTC = 10240
RPTD = NTC // NS
RBLK = 1280
GBLK = NTC // RBLK
EPAD = NW * 79 * 128
NEB = 79

_mesh = plsc.VectorSubcoreMesh(core_axis_name="c", subcore_axis_name="s")


@functools.partial(
    pl.kernel, mesh=_mesh,
    out_type=jax.ShapeDtypeStruct((NC * NTC,), jnp.float32),
    scratch_types=[
        pltpu.VMEM((NEB, 128), jnp.int32),
        pltpu.VMEM((128,), jnp.float32),
        pltpu.VMEM_SHARED((NTC,), jnp.float32),
        pltpu.SemaphoreType.DMA,
        pltpu.SemaphoreType.DMA,
    ],
)
def _deg_kernel(dst_hbm, ones_hbm, zeros1_hbm, out_hbm, idx_v, ones_v, hist_sh, dsem0, dsem1):
    c = lax.axis_index("c")
    s = lax.axis_index("s")
    w = c * NS + s
    pltpu.sync_copy(zeros1_hbm.at[pl.ds(s * RPTD, RPTD)], hist_sh.at[pl.ds(s * RPTD, RPTD)])
    pltpu.sync_copy(ones_hbm, ones_v)
    pltpu.sync_copy(dst_hbm.at[w], idx_v)
    plsc.subcore_barrier()

    def _dstart(j, sem):
        pltpu.async_copy(ones_v, hist_sh.at[idx_v.at[j]], sem, add=True)

    def _dwait(sem):
        pltpu.make_async_copy(ones_v, hist_sh.at[idx_v.at[0]], sem).wait()

    _dstart(0, dsem0)

    def body(i, carry):
        j = 2 * i
        _dstart(j + 1, dsem1)
        _dwait(dsem0)
        _dstart(j + 2, dsem0)
        _dwait(dsem1)
        return carry

    lax.fori_loop(0, (NEB - 1) // 2, body, 0)
    _dwait(dsem0)
    plsc.subcore_barrier()
    pltpu.sync_copy(hist_sh.at[pl.ds(s * RPTD, RPTD)],
                    out_hbm.at[pl.ds(c * NTC + s * RPTD, RPTD)])


@functools.partial(
    pl.kernel, mesh=_mesh,
    out_type=jax.ShapeDtypeStruct((NC, NTC, H), jnp.float32),
    scratch_types=[
        pltpu.VMEM((NEB, 128), jnp.int32),
        pltpu.VMEM((2, 128), jnp.int32),
        pltpu.VMEM((2, 128, H), jnp.float32),
        pltpu.VMEM_SHARED((NACC, H), jnp.float32),
        pltpu.SemaphoreType.DMA,
        pltpu.SemaphoreType.DMA,
    ],
)
def _scat_kernel(y_hbm, srcw_hbm, dstw_hbm, zeros2_hbm, out_hbm,
                 src_v, dstc_v, rows_v, acc_sh, gsem0, gsem1):
    c = lax.axis_index("c")
    s = lax.axis_index("s")
    w = c * NS + s
    @pl.when(c == 0)
    def _():
        pltpu.sync_copy(y_hbm.at[pl.ds(s * RPT, RPT)], acc_sh.at[pl.ds(s * RPT, RPT)])

    @pl.when(c != 0)
    def _():
        pltpu.sync_copy(zeros2_hbm.at[pl.ds(s * RPT, RPT)], acc_sh.at[pl.ds(s * RPT, RPT)])

    pltpu.sync_copy(srcw_hbm.at[w], src_v)
    plsc.subcore_barrier()

    def _start(j, b, sem):
        pltpu.async_copy(y_hbm.at[src_v.at[j]], rows_v.at[b], sem)
        pltpu.async_copy(dstw_hbm.at[w, j], dstc_v.at[b], sem)

    def _wait(b, sem):
        pltpu.make_async_copy(y_hbm.at[src_v.at[0]], rows_v.at[b], sem).wait()
        pltpu.make_async_copy(dstw_hbm.at[0, 0], dstc_v.at[b], sem).wait()

    def _scat(b):
        pltpu.sync_copy(rows_v.at[b], acc_sh.at[dstc_v.at[b]], add=True)

    _start(0, 0, gsem0)

    def body(i, carry):
        j = 2 * i
        _start(j + 1, 1, gsem1)
        _wait(0, gsem0)
        _scat(0)
        _start(j + 2, 0, gsem0)
        _wait(1, gsem1)
        _scat(1)
        return carry

    lax.fori_loop(0, (NEB - 1) // 2, body, 0)
    _wait(0, gsem0)
    _scat(0)
    plsc.subcore_barrier()
    pltpu.sync_copy(acc_sh.at[pl.ds(s * RPT, RPT)], out_hbm.at[c, pl.ds(s * RPT, RPT)])


def _stage0_body(deg_ref, x_ref, w_ref, y_ref, dinv_ref):
    deg = deg_ref[0, :] + deg_ref[1, :] + 1.0
    dcol = lax.rsqrt(deg)[:, None]
    dinv_ref[...] = dcol
    xw = jnp.dot(x_ref[...], w_ref[...], preferred_element_type=jnp.float32)
    y_ref[...] = xw * dcol


_stage0 = pl.pallas_call(
    _stage0_body,
    grid=(GBLK,),
    in_specs=[
        pl.BlockSpec((2, RBLK), lambda i: (0, i)),
        pl.BlockSpec((RBLK, 128), lambda i: (i, 0)),
        pl.BlockSpec((128, 128), lambda i: (0, 0)),
    ],
    out_specs=[
        pl.BlockSpec((RBLK, 128), lambda i: (i, 0)),
        pl.BlockSpec((RBLK, 1), lambda i: (i, 0)),
    ],
    out_shape=[
        jax.ShapeDtypeStruct((NTC, H), jnp.float32),
        jax.ShapeDtypeStruct((NTC, 1), jnp.float32),
    ],
)


def _ln_relu(gcn, g, be):
    a = jnp.maximum(gcn, 0.0)
    m = jnp.mean(a, axis=-1, keepdims=True)
    v = jnp.mean((a - m) ** 2, axis=-1, keepdims=True)
    return (a - m) / jnp.sqrt(v + 1e-5) * g[None, :] + be[None, :]


def _mid_body(p_ref, dinv_ref, b_ref, g_ref, be_ref, w_ref, o_ref):
    d = dinv_ref[...]
    z = p_ref[0] + p_ref[1]
    gcn = z * d + b_ref[...][None, :]
    h = _ln_relu(gcn, g_ref[...], be_ref[...])
    o_ref[...] = jnp.dot(h, w_ref[...], preferred_element_type=jnp.float32) * d


_stage_mid = pl.pallas_call(
    _mid_body,
    grid=(GBLK,),
    in_specs=[
        pl.BlockSpec((2, RBLK, 128), lambda i: (0, i, 0)),
        pl.BlockSpec((RBLK, 1), lambda i: (i, 0)),
        pl.BlockSpec((128,), lambda i: (0,)),
        pl.BlockSpec((128,), lambda i: (0,)),
        pl.BlockSpec((128,), lambda i: (0,)),
        pl.BlockSpec((128, 128), lambda i: (0, 0)),
    ],
    out_specs=pl.BlockSpec((RBLK, 128), lambda i: (i, 0)),
    out_shape=jax.ShapeDtypeStruct((NTC, H), jnp.float32),
)


def _s3_body(p_ref, dinv_ref, b_ref, g_ref, be_ref,
             wp0_ref, bp0_ref, wp1_ref, bp1_ref, wp2_ref, bp2_ref,
             emb_ref, lp_ref):
    d = dinv_ref[...]
    z = p_ref[0] + p_ref[1]
    emb = z * d + b_ref[...][None, :]
    emb_ref[...] = emb
    h = _ln_relu(emb, g_ref[...], be_ref[...])
    t = jnp.dot(h, wp0_ref[...], preferred_element_type=jnp.float32) + bp0_ref[...][None, :]
    t = jnp.dot(t, wp1_ref[...], preferred_element_type=jnp.float32) + bp1_ref[...][None, :]
    t = jnp.dot(t, wp2_ref[...], preferred_element_type=jnp.float32) + bp2_ref[...][None, :]
    mx = jnp.max(t, axis=-1, keepdims=True)
    lse = mx + jnp.log(jnp.sum(jnp.exp(t - mx), axis=-1, keepdims=True))
    lp_ref[...] = t - lse


_stage3 = pl.pallas_call(
    _s3_body,
    grid=(GBLK,),
    in_specs=[
        pl.BlockSpec((2, RBLK, 128), lambda i: (0, i, 0)),
        pl.BlockSpec((RBLK, 1), lambda i: (i, 0)),
        pl.BlockSpec((128,), lambda i: (0,)),
        pl.BlockSpec((128,), lambda i: (0,)),
        pl.BlockSpec((128,), lambda i: (0,)),
        pl.BlockSpec((128, DH), lambda i: (0, 0)),
        pl.BlockSpec((DH,), lambda i: (0,)),
        pl.BlockSpec((DH, DH // 2), lambda i: (0, 0)),
        pl.BlockSpec((DH // 2,), lambda i: (0,)),
        pl.BlockSpec((DH // 2, OUT), lambda i: (0, 0)),
        pl.BlockSpec((OUT,), lambda i: (0,)),
    ],
    out_specs=[
        pl.BlockSpec((RBLK, 128), lambda i: (i, 0)),
        pl.BlockSpec((RBLK, OUT), lambda i: (i, 0)),
    ],
    out_shape=[
        jax.ShapeDtypeStruct((N, H), jnp.float32),
        jax.ShapeDtypeStruct((N, OUT), jnp.float32),
    ],
)


def kernel(x, edge_index, W1, b1, g1, be1, Wc0, bc0, gc0, bec0,
           Wc1, bc1, gc1, bec1, Wp0, bp0, Wp1, bp1, Wp2, bp2):
    src = edge_index[0]
    dst = edge_index[1]
    pad = (N + (jnp.arange(EPAD - E, dtype=jnp.int32) % (NACC - N))).astype(jnp.int32)
    srcp = jnp.concatenate([src, pad]).reshape(NW, NEB, 128)
    dstp128 = jnp.concatenate([dst, pad]).reshape(NW, NEB, 128)
    ones128 = jnp.ones((128,), jnp.float32)
    zeros1 = jnp.zeros((NTC,), jnp.float32)
    zeros2 = jnp.zeros((NACC, H), jnp.float32)

    degp = _deg_kernel(dstp128, ones128, zeros1)
    xpad = jnp.pad(x, ((0, NTC - N), (0, 0)))
    y0, dinv = _stage0(degp.reshape(NC, NTC), xpad, W1)
    p0 = _scat_kernel(y0, srcp, dstp128, zeros2)
    y1 = _stage_mid(p0, dinv, b1, g1, be1, Wc0)
    p1 = _scat_kernel(y1, srcp, dstp128, zeros2)
    y2 = _stage_mid(p1, dinv, bc0, gc0, bec0, Wc1)
    p2 = _scat_kernel(y2, srcp, dstp128, zeros2)
    emb, logp = _stage3(p2, dinv, bc1, gc1, bec1,
                        Wp0, bp0, Wp1, bp1, Wp2, bp2)
    return emb, logp
